# baseline (speedup 1.0000x reference)
import jax
import jax.numpy as jnp
import numpy as np
from jax.experimental import pallas as pl

_T = 25
_CP = 66
_CO = 96
_BB = 32


def _channel_maps():
    used = np.array([6, 7, 8, 9, 10, 11, 12, 13, 14, 15, 16, 17, 21, 22, 23,
                     24, 25, 26, 27, 28, 29, 30, 31, 32, 36, 37, 38, 39, 40,
                     41, 42, 43, 44, 45, 46, 47, 51, 52, 53, 54, 55, 56, 57,
                     58, 59, 63, 64, 65, 66, 67, 68, 75, 76, 77, 78, 79, 80,
                     81, 82, 83, 87, 88, 89, 90, 91, 92])

    def j2i(x):
        return np.concatenate((x * 3, x * 3 + 1, x * 3 + 2))

    idx_copy = j2i(np.array([0, 1, 6, 11]))
    idx_ignore = j2i(np.array([16, 20, 23, 24, 28, 31]))
    idx_equal = j2i(np.array([13, 19, 22, 13, 27, 30]))

    src = np.full(_CO, -1)
    idx = np.full(_CO, -1)
    src[used] = 0
    idx[used] = np.arange(_CP)
    src[idx_copy] = 1
    idx[idx_copy] = idx_copy
    src[idx_ignore] = src[idx_equal]
    idx[idx_ignore] = idx[idx_equal]

    mp = np.zeros((_CP, _CO), np.float32)
    mo = np.zeros((_CO, _CO), np.float32)
    for o in range(_CO):
        if src[o] == 0:
            mp[idx[o], o] = 1.0
        else:
            mo[idx[o], o] = 1.0
    return mp, mo


_MP, _MO = _channel_maps()
_R = np.zeros((_BB * _T, _BB), np.float32)
_R[np.arange(_BB * _T), np.arange(_BB * _T) // _T] = 1.0


def _body(obs_ref, pred_ref, mp_ref, mo_ref, r_ref, out_ref):
    obs_part = jnp.dot(obs_ref[...], mo_ref[...],
                       preferred_element_type=jnp.float32)
    spread = jnp.dot(r_ref[...], obs_part,
                     preferred_element_type=jnp.float32)
    pred_part = jnp.dot(pred_ref[...], mp_ref[...],
                        preferred_element_type=jnp.float32)
    out_ref[...] = pred_part + spread


def kernel(observed_pose, pred_pose):
    B, T, C = pred_pose.shape
    obs_last = observed_pose[:, -1, :]
    pred_rows = pred_pose.reshape(B * T, C)
    rows_per_block = _BB * T
    grid = (B // _BB,)
    out = pl.pallas_call(
        _body,
        grid=grid,
        in_specs=[
            pl.BlockSpec((_BB, _CO), lambda i: (i, 0)),
            pl.BlockSpec((rows_per_block, C), lambda i: (i, 0)),
            pl.BlockSpec((C, _CO), lambda i: (0, 0)),
            pl.BlockSpec((_CO, _CO), lambda i: (0, 0)),
            pl.BlockSpec((rows_per_block, _BB), lambda i: (0, 0)),
        ],
        out_specs=pl.BlockSpec((rows_per_block, _CO), lambda i: (i, 0)),
        out_shape=jax.ShapeDtypeStruct((B * T, _CO), pred_pose.dtype),
    )(obs_last, pred_rows, jnp.asarray(_MP), jnp.asarray(_MO),
      jnp.asarray(_R))
    return out.reshape(B, T, _CO)


# speedup vs baseline: 1.2442x; 1.2442x over previous
import jax
import jax.numpy as jnp
import numpy as np
from jax.experimental import pallas as pl

_T = 25
_CP = 66
_CO = 96
_BB = 128


def _channel_maps():
    used = np.array([6, 7, 8, 9, 10, 11, 12, 13, 14, 15, 16, 17, 21, 22, 23,
                     24, 25, 26, 27, 28, 29, 30, 31, 32, 36, 37, 38, 39, 40,
                     41, 42, 43, 44, 45, 46, 47, 51, 52, 53, 54, 55, 56, 57,
                     58, 59, 63, 64, 65, 66, 67, 68, 75, 76, 77, 78, 79, 80,
                     81, 82, 83, 87, 88, 89, 90, 91, 92])

    def j2i(x):
        return np.concatenate((x * 3, x * 3 + 1, x * 3 + 2))

    idx_copy = j2i(np.array([0, 1, 6, 11]))
    idx_ignore = j2i(np.array([16, 20, 23, 24, 28, 31]))
    idx_equal = j2i(np.array([13, 19, 22, 13, 27, 30]))

    src = np.full(_CO, -1)
    idx = np.full(_CO, -1)
    src[used] = 0
    idx[used] = np.arange(_CP)
    src[idx_copy] = 1
    idx[idx_copy] = idx_copy
    src[idx_ignore] = src[idx_equal]
    idx[idx_ignore] = idx[idx_equal]

    mp = np.zeros((_CP, _CO), np.float32)
    mo = np.zeros((_CO, _CO), np.float32)
    for o in range(_CO):
        if src[o] == 0:
            mp[idx[o], o] = 1.0
        else:
            mo[idx[o], o] = 1.0
    return mp, mo


_MP, _MO = _channel_maps()
_R = np.zeros((_BB * _T, _BB), np.float32)
_R[np.arange(_BB * _T), np.arange(_BB * _T) // _T] = 1.0


def _body(obs_ref, pred_ref, mp_ref, mo_ref, r_ref, out_ref):
    obs_part = jnp.dot(obs_ref[...], mo_ref[...],
                       preferred_element_type=jnp.float32)
    spread = jnp.dot(r_ref[...], obs_part,
                     preferred_element_type=jnp.float32)
    pred_part = jnp.dot(pred_ref[...], mp_ref[...],
                        preferred_element_type=jnp.float32)
    out_ref[...] = pred_part + spread


def kernel(observed_pose, pred_pose):
    B, T, C = pred_pose.shape
    obs_last = observed_pose[:, -1, :]
    pred_rows = pred_pose.reshape(B * T, C)
    rows_per_block = _BB * T
    grid = (B // _BB,)
    out = pl.pallas_call(
        _body,
        grid=grid,
        in_specs=[
            pl.BlockSpec((_BB, _CO), lambda i: (i, 0)),
            pl.BlockSpec((rows_per_block, C), lambda i: (i, 0)),
            pl.BlockSpec((C, _CO), lambda i: (0, 0)),
            pl.BlockSpec((_CO, _CO), lambda i: (0, 0)),
            pl.BlockSpec((rows_per_block, _BB), lambda i: (0, 0)),
        ],
        out_specs=pl.BlockSpec((rows_per_block, _CO), lambda i: (i, 0)),
        out_shape=jax.ShapeDtypeStruct((B * T, _CO), pred_pose.dtype),
    )(obs_last, pred_rows, jnp.asarray(_MP), jnp.asarray(_MO),
      jnp.asarray(_R))
    return out.reshape(B, T, _CO)
